# speedup vs baseline: 1.6137x; 1.6137x over previous
import functools

import jax
import jax.numpy as jnp
from jax import lax
from jax.experimental import pallas as pl
from jax.experimental.pallas import tpu as pltpu
from jax.experimental.pallas import tpu_sc as plsc

VOCAB = 100000
HIDDEN = 1024
BATCH = 4
SEQ = 4096

_B = BATCH * SEQ


def _make_gather(V, D, B):
    info = plsc.get_sparse_core_info()
    NC, NS = info.num_cores, info.num_subcores
    NW = NC * NS
    assert B % NW == 0
    b_per_w = B // NW
    CHUNK = 32
    n_steps = b_per_w // CHUNK
    assert n_steps % 2 == 0
    mesh = plsc.VectorSubcoreMesh(core_axis_name="c", subcore_axis_name="s")

    @functools.partial(
        pl.kernel,
        mesh=mesh,
        out_type=jax.ShapeDtypeStruct((B, D), jnp.float32),
        scratch_types=[
            pltpu.VMEM((b_per_w,), jnp.int32),
            pltpu.VMEM((CHUNK, D), jnp.float32),
            pltpu.VMEM((CHUNK, D), jnp.float32),
            pltpu.SemaphoreType.DMA,
            pltpu.SemaphoreType.DMA,
        ],
    )
    def gather_kernel(ids_hbm, table_hbm, out_hbm, idx_v, rows0, rows1, sem0, sem1):
        wid = lax.axis_index("s") * NC + lax.axis_index("c")
        base = wid * b_per_w
        pltpu.sync_copy(ids_hbm.at[pl.ds(base, b_per_w)], idx_v)

        bufs = (rows0, rows1)
        sems = (sem0, sem1)

        pltpu.async_copy(table_hbm.at[idx_v.at[pl.ds(0, CHUNK)]], bufs[0], sems[0])

        def pair(i, carry):
            for p in range(2):
                g = i * 2 + p
                pltpu.make_async_copy(
                    table_hbm.at[idx_v.at[pl.ds(g * CHUNK, CHUNK)]],
                    bufs[p],
                    sems[p],
                ).wait()
                nxt = g + 1

                @pl.when(nxt < n_steps)
                def _():
                    pltpu.async_copy(
                        table_hbm.at[idx_v.at[pl.ds(nxt * CHUNK, CHUNK)]],
                        bufs[(p + 1) % 2],
                        sems[(p + 1) % 2],
                    )

                pltpu.sync_copy(bufs[p], out_hbm.at[pl.ds(base + g * CHUNK, CHUNK)])
            return carry

        lax.fori_loop(0, n_steps // 2, pair, 0)

    return gather_kernel


_gather = _make_gather(VOCAB, HIDDEN, _B)


@jax.jit
def kernel(input_ids, labels, embed_weight):
    del labels
    ids = input_ids.reshape(-1).astype(jnp.int32)
    out = _gather(ids, embed_weight)
    return out.reshape(BATCH, SEQ, HIDDEN)
